# baseline (speedup 1.0000x reference)
import functools

import jax
import jax.numpy as jnp
from jax import lax
from jax.experimental import pallas as pl
from jax.experimental.pallas import tpu as pltpu
from jax.experimental.pallas import tpu_sc as plsc

_K = 64
_M = 128
_N = 32768

_NC, _NS, _L = 2, 16, 16
_NW = _NC * _NS
_RPW = _M // _NW
_NBUF = 3
_NB = 2048
_HV = _NB // _L
_SV = _HV // _L
_CAP = 4096
_LOWB = 21


def _sortable(v):
    u = lax.bitcast_convert_type(v, jnp.uint32)
    neg = (u >> jnp.uint32(31)) > jnp.uint32(0)
    return jnp.where(neg, ~u, u | jnp.uint32(0x80000000))


def _sc_body(x_hbm, out_hbm, row_a, row_b, row_c, hist_v, sums_v, cand_v,
             sin_a, sin_b, sin_c, sout_a, sout_b, sout_c):
    c = lax.axis_index("c")
    s = lax.axis_index("s")
    wid = s * _NC + c
    base = wid * _RPW
    lanes = lax.iota(jnp.int32, _L)
    bufs = (row_a, row_b, row_c)
    sins = (sin_a, sin_b, sin_c)
    souts = (sout_a, sout_b, sout_c)

    def hist_pass(buf):
        @plsc.parallel_loop(0, _NB, step=_L, unroll=8)
        def _(i):
            hist_v[pl.ds(i, _L)] = jnp.zeros((_L,), jnp.int32)

        ones = jnp.ones((_L,), jnp.int32)

        @plsc.parallel_loop(0, _N, step=_L, unroll=8)
        def _(i):
            su = _sortable(buf[pl.ds(i, _L)])
            b = (su >> jnp.uint32(_LOWB)).astype(jnp.int32)
            plsc.addupdate_scatter(hist_v, [b], ones)

    def find_bucket(t_lvl, r):
        @plsc.parallel_loop(0, _SV, unroll=1)
        def _(i):
            sums_v[pl.ds(i * _L, _L)] = jnp.zeros((_L,), jnp.int32)

        lane0 = lanes == 0

        @plsc.parallel_loop(0, _HV, unroll=4)
        def _(i):
            hv = hist_v[pl.ds(i * _L, _L)]
            sv = jnp.full((_L,), jnp.sum(hv, axis=0))
            iv = jnp.full((_L,), i, jnp.int32)
            plsc.addupdate_scatter(sums_v, [iv], sv, mask=lane0)

        lim = t_lvl - r
        pre = jnp.int32(0)
        pres = []
        nhits = jnp.int32(0)
        for i in range(_SV):
            sv = sums_v[pl.ds(i * _L, _L)]
            cs = plsc.cumsum(sv)
            pres.append(pre)
            hit = (pre + cs - sv) <= lim
            nhits = nhits + plsc.all_reduce_population_count(hit)[0]
            pre = pre + cs[_L - 1]
        gidx = nhits - 1
        gv = gidx // _L
        gl = gidx % _L
        pre_g = jnp.int32(0)
        for i in range(_SV):
            pre_g = jnp.where(gv == i, pres[i], pre_g)
        sv = sums_v[pl.ds(gv * _L, _L)]
        cs = plsc.cumsum(sv)
        excl = pre_g + cs - sv
        pre_grp = jnp.sum(jnp.where(lanes == gl, excl, 0), axis=0)

        hv = hist_v[pl.ds(gidx * _L, _L)]
        hcs = plsc.cumsum(hv)
        hexcl = pre_grp + hcs - hv
        hhit = hexcl <= lim
        lsel = plsc.all_reduce_population_count(hhit)[0] - 1
        s_sel = jnp.sum(jnp.where(lanes == lsel, hv, 0), axis=0)
        p_sel = jnp.sum(jnp.where(lanes == lsel, hexcl, 0), axis=0)
        bucket = gidx * _L + lsel
        r_next = r - (t_lvl - p_sel - s_sel)
        return bucket, s_sel, r_next

    def compact(buf, b1):
        b1u = b1.astype(jnp.uint32)
        lowmask = jnp.uint32((1 << _LOWB) - 1)

        @plsc.parallel_loop(0, _N, step=_L, unroll=4, carry=jnp.int32(0))
        def off(i, off):
            su = _sortable(buf[pl.ds(i, _L)])
            m = (su >> jnp.uint32(_LOWB)) == b1u
            low = (su & lowmask).astype(jnp.int32)
            safe = jnp.minimum(off, jnp.int32(_CAP))
            plsc.store_compressed(cand_v.at[pl.ds(safe, _L)], low, mask=m)
            return off + plsc.all_reduce_population_count(m)[0]

        safe = jnp.minimum(off, jnp.int32(_CAP))
        cand_v[pl.ds(safe, _L)] = jnp.zeros((_L,), jnp.int32)
        return off

    def refine_cand(ncand, r):
        nv = (ncand + _L - 1) // _L

        def outer(b, v):
            cand_t = v | (jnp.int32(1) << (jnp.int32(_LOWB - 1) - b))

            def inner(j, acc):
                lv = cand_v[pl.ds(j * _L, _L)]
                m = lv >= cand_t
                return acc + plsc.all_reduce_population_count(m)[0]

            cnt = lax.fori_loop(0, nv, inner, jnp.int32(0))
            return jnp.where(cnt >= r, cand_t, v)

        return lax.fori_loop(0, _LOWB, outer, jnp.int32(0))

    def refine_row(buf, b1):
        hi = b1.astype(jnp.uint32) << jnp.uint32(_LOWB)

        def outer(b, v):
            cand_t = v | (jnp.int32(1) << (jnp.int32(_LOWB - 1) - b))
            tfull = hi | cand_t.astype(jnp.uint32)

            def inner(j, acc):
                su = _sortable(buf[pl.ds(j * _L, _L)])
                m = su >= tfull
                return acc + plsc.all_reduce_population_count(m)[0]

            cnt = lax.fori_loop(0, _N // _L, inner, jnp.int32(0))
            return jnp.where(cnt >= jnp.int32(_K), cand_t, v)

        return lax.fori_loop(0, _LOWB, outer, jnp.int32(0))

    def threshold_vec(buf):
        hist_pass(buf)
        b1, s_sel, r = find_bucket(jnp.int32(_N), jnp.int32(_K))
        ncand = compact(buf, b1)
        vlow = lax.cond(
            ncand <= jnp.int32(_CAP),
            lambda: refine_cand(ncand, r),
            lambda: refine_row(buf, b1),
        )
        tsu = (b1 << _LOWB) | vlow
        tvec = jnp.full((_L,), tsu.astype(jnp.uint32))
        pos = (tvec >> jnp.uint32(31)) > jnp.uint32(0)
        uvec = jnp.where(pos, tvec & jnp.uint32(0x7FFFFFFF), ~tvec)
        return lax.bitcast_convert_type(uvec, jnp.float32)

    def mask_pass(buf, thr):
        zero = jnp.zeros((_L,), jnp.float32)

        @plsc.parallel_loop(0, _N, step=_L, unroll=8)
        def _(i):
            v = buf[pl.ds(i, _L)]
            buf[pl.ds(i, _L)] = jnp.where(v >= thr, v, zero)

    in_copies = [None] * _RPW
    out_copies = [None] * _RPW
    in_copies[0] = pltpu.async_copy(x_hbm.at[base], bufs[0], sins[0])
    in_copies[1] = pltpu.async_copy(x_hbm.at[base + 1], bufs[1], sins[1])
    for k in range(_RPW):
        buf = bufs[k % _NBUF]
        in_copies[k].wait()
        thr = threshold_vec(buf)
        if k + 2 < _RPW:
            if k >= 1:
                out_copies[k - 1].wait()
            in_copies[k + 2] = pltpu.async_copy(
                x_hbm.at[base + k + 2], bufs[(k + 2) % _NBUF],
                sins[(k + 2) % _NBUF])
        mask_pass(buf, thr)
        out_copies[k] = pltpu.async_copy(buf, out_hbm.at[base + k],
                                         souts[k % _NBUF])
    out_copies[_RPW - 2].wait()
    out_copies[_RPW - 1].wait()


@jax.jit
def kernel(x):
    m, n = x.shape
    run = pl.kernel(
        _sc_body,
        out_type=jax.ShapeDtypeStruct((m, n), jnp.float32),
        mesh=plsc.VectorSubcoreMesh(core_axis_name="c", subcore_axis_name="s"),
        compiler_params=pltpu.CompilerParams(needs_layout_passes=False),
        scratch_types=[
            pltpu.VMEM((_N,), jnp.float32),
            pltpu.VMEM((_N,), jnp.float32),
            pltpu.VMEM((_N,), jnp.float32),
            pltpu.VMEM((_NB,), jnp.int32),
            pltpu.VMEM((_HV,), jnp.int32),
            pltpu.VMEM((_CAP + 2 * _L,), jnp.int32),
            pltpu.SemaphoreType.DMA,
            pltpu.SemaphoreType.DMA,
            pltpu.SemaphoreType.DMA,
            pltpu.SemaphoreType.DMA,
            pltpu.SemaphoreType.DMA,
            pltpu.SemaphoreType.DMA,
        ],
    )
    return run(x)
